# baseline (speedup 1.0000x reference)
import jax
import jax.numpy as jnp
from jax.experimental import pallas as pl
from jax.experimental.pallas import tpu as pltpu

_LAMBD = 0.5
_EPS = 1e-10
_C = 512
_R = 5000


def _partials_kernel(o_ref, l_ref, ssq_ref, tp_ref, nl_ref):
    o = o_ref[...]
    l = l_ref[...]
    d = o - l
    ssq_ref[0, 0, :] = jnp.sum(d * d, axis=0)
    tp_ref[0, 0, :] = jnp.sum(jnp.where(o > 0.5, l, 0.0), axis=0)
    nl_ref[0, 0, :] = jnp.sum(l, axis=0)


def kernel(outputs, labels):
    n = outputs.size
    rows = n // _C
    g = rows // _R
    o2 = outputs.reshape(rows, _C)
    l2 = labels.reshape(rows, _C)
    out_sds = jax.ShapeDtypeStruct((g, 1, _C), jnp.float32)
    ssq_p, tp_p, nl_p = pl.pallas_call(
        _partials_kernel,
        grid=(g,),
        in_specs=[
            pl.BlockSpec((_R, _C), lambda i: (i, 0)),
            pl.BlockSpec((_R, _C), lambda i: (i, 0)),
        ],
        out_specs=[
            pl.BlockSpec((1, 1, _C), lambda i: (i, 0, 0)),
            pl.BlockSpec((1, 1, _C), lambda i: (i, 0, 0)),
            pl.BlockSpec((1, 1, _C), lambda i: (i, 0, 0)),
        ],
        out_shape=[out_sds, out_sds, out_sds],
        compiler_params=pltpu.CompilerParams(
            dimension_semantics=("parallel",),
            vmem_limit_bytes=56 * 1024 * 1024,
        ),
        name="detection_loss_partials",
    )(o2, l2)

    ssq = jnp.sum(ssq_p)
    tp = jnp.sum(tp_p)
    nl = jnp.sum(nl_p)
    fn = nl - tp
    mse = ssq / jnp.float32(n)
    coeff = jnp.where(
        (tp == 0) & (fn == 0),
        jnp.float32(1.0),
        jnp.where(tp == 0, jnp.float32(0.0), tp / (tp + fn)),
    )
    cost = jax.lax.stop_gradient(-jnp.log(coeff + _EPS))
    return mse + _LAMBD * cost


# speedup vs baseline: 55.9266x; 55.9266x over previous
import jax
import jax.numpy as jnp
from jax.experimental import pallas as pl
from jax.experimental.pallas import tpu as pltpu

_LAMBD = 0.5
_EPS = 1e-10
_BC = 32768


def _partials_kernel(o_ref, l_ref, ssq_ref, tp_ref, nl_ref, *, total_cols):
    i = pl.program_id(0)
    o = o_ref[...]
    l = l_ref[...]
    d = o - l
    ssq = jnp.sum(d * d, axis=0, keepdims=True)
    tp = jnp.sum(jnp.where(o > 0.5, l, 0.0), axis=0, keepdims=True)
    nl = jnp.sum(l, axis=0, keepdims=True)
    col = jax.lax.broadcasted_iota(jnp.int32, ssq.shape, 1) + i * ssq.shape[1]
    valid = col < total_cols
    zero = jnp.zeros_like(ssq)
    ssq_ref[0] = jnp.where(valid, ssq, zero)
    tp_ref[0] = jnp.where(valid, tp, zero)
    nl_ref[0] = jnp.where(valid, nl, zero)


def kernel(outputs, labels):
    import functools

    n = outputs.size
    rows, cols = outputs.shape
    g = (cols + _BC - 1) // _BC
    out_sds = jax.ShapeDtypeStruct((g, 1, _BC), jnp.float32)
    ssq_p, tp_p, nl_p = pl.pallas_call(
        functools.partial(_partials_kernel, total_cols=cols),
        grid=(g,),
        in_specs=[
            pl.BlockSpec((rows, _BC), lambda i: (0, i)),
            pl.BlockSpec((rows, _BC), lambda i: (0, i)),
        ],
        out_specs=[
            pl.BlockSpec((1, 1, _BC), lambda i: (i, 0, 0)),
            pl.BlockSpec((1, 1, _BC), lambda i: (i, 0, 0)),
            pl.BlockSpec((1, 1, _BC), lambda i: (i, 0, 0)),
        ],
        out_shape=[out_sds, out_sds, out_sds],
        compiler_params=pltpu.CompilerParams(
            dimension_semantics=("parallel",),
            vmem_limit_bytes=56 * 1024 * 1024,
        ),
        name="detection_loss_partials",
    )(outputs, labels)

    ssq = jnp.sum(ssq_p)
    tp = jnp.sum(tp_p)
    nl = jnp.sum(nl_p)
    fn = nl - tp
    mse = ssq / jnp.float32(n)
    coeff = jnp.where(
        (tp == 0) & (fn == 0),
        jnp.float32(1.0),
        jnp.where(tp == 0, jnp.float32(0.0), tp / (tp + fn)),
    )
    cost = jax.lax.stop_gradient(-jnp.log(coeff + _EPS))
    return mse + _LAMBD * cost


# speedup vs baseline: 69.4444x; 1.2417x over previous
import functools

import jax
import jax.numpy as jnp
from jax.experimental import pallas as pl
from jax.experimental.pallas import tpu as pltpu

_LAMBD = 0.5
_EPS = 1e-10
_BC = 32768


def _loss_kernel(o_ref, l_ref, out_ref, ssq_ref, tp_ref, nl_ref, *,
                 total_cols, total_n, grid_len):
    i = pl.program_id(0)

    @pl.when(i == 0)
    def _init():
        ssq_ref[...] = jnp.zeros_like(ssq_ref)
        tp_ref[...] = jnp.zeros_like(tp_ref)
        nl_ref[...] = jnp.zeros_like(nl_ref)

    o = o_ref[...]
    l = l_ref[...]
    d = o - l
    bc = ssq_ref.shape[1]
    col = jax.lax.broadcasted_iota(jnp.int32, (1, bc), 1) + i * bc
    valid = col < total_cols
    zero = jnp.zeros((1, bc), jnp.float32)
    ssq_ref[...] += jnp.where(valid, jnp.sum(d * d, axis=0, keepdims=True), zero)
    tp_ref[...] += jnp.where(
        valid, jnp.sum(jnp.where(o > 0.5, l, 0.0), axis=0, keepdims=True), zero)
    nl_ref[...] += jnp.where(valid, jnp.sum(l, axis=0, keepdims=True), zero)

    @pl.when(i == grid_len - 1)
    def _finish():
        ssq = jnp.sum(ssq_ref[...])
        tp = jnp.sum(tp_ref[...])
        nl = jnp.sum(nl_ref[...])
        fn = nl - tp
        mse = ssq / jnp.float32(total_n)
        coeff = jnp.where(
            (tp == 0) & (fn == 0),
            jnp.float32(1.0),
            jnp.where(tp == 0, jnp.float32(0.0), tp / (tp + fn)),
        )
        out_ref[0, 0] = mse + _LAMBD * (-jnp.log(coeff + _EPS))


def kernel(outputs, labels):
    rows, cols = outputs.shape
    g = (cols + _BC - 1) // _BC
    body = functools.partial(
        _loss_kernel, total_cols=cols, total_n=outputs.size, grid_len=g)
    out = pl.pallas_call(
        body,
        grid=(g,),
        in_specs=[
            pl.BlockSpec((rows, _BC), lambda i: (0, i)),
            pl.BlockSpec((rows, _BC), lambda i: (0, i)),
        ],
        out_specs=pl.BlockSpec(memory_space=pltpu.SMEM),
        out_shape=jax.ShapeDtypeStruct((1, 1), jnp.float32),
        scratch_shapes=[
            pltpu.VMEM((1, _BC), jnp.float32),
            pltpu.VMEM((1, _BC), jnp.float32),
            pltpu.VMEM((1, _BC), jnp.float32),
        ],
        compiler_params=pltpu.CompilerParams(
            dimension_semantics=("arbitrary",),
            vmem_limit_bytes=56 * 1024 * 1024,
        ),
        name="detection_loss_fused",
    )(outputs, labels)
    return out[0, 0]


# speedup vs baseline: 69.6923x; 1.0036x over previous
import functools

import jax
import jax.numpy as jnp
from jax.experimental import pallas as pl
from jax.experimental.pallas import tpu as pltpu

_LAMBD = 0.5
_EPS = 1e-10
_BC = 32768


def _loss_kernel(o_ref, l_ref, out_ref, ssq_ref, tp_ref, nl_ref, *,
                 total_cols, total_n, grid_len):
    i = pl.program_id(0)

    @pl.when(i == 0)
    def _init():
        ssq_ref[...] = jnp.zeros_like(ssq_ref)
        tp_ref[...] = jnp.zeros_like(tp_ref)
        nl_ref[...] = jnp.zeros_like(nl_ref)

    o = o_ref[...]
    l = l_ref[...]
    d = o - l
    bc = ssq_ref.shape[1]
    col = jax.lax.broadcasted_iota(jnp.int32, (1, bc), 1) + i * bc
    valid = col < total_cols
    zero = jnp.zeros((1, bc), jnp.float32)
    ssq_ref[...] += jnp.where(valid, jnp.sum(d * d, axis=0, keepdims=True), zero)
    tp_ref[...] += jnp.where(
        valid, jnp.sum(jnp.where(o > 0.5, l, 0.0), axis=0, keepdims=True), zero)
    nl_ref[...] += jnp.where(valid, jnp.sum(l, axis=0, keepdims=True), zero)

    @pl.when(i == grid_len - 1)
    def _finish():
        ssq = jnp.sum(ssq_ref[...])
        tp = jnp.sum(tp_ref[...])
        nl = jnp.sum(nl_ref[...])
        fn = nl - tp
        mse = ssq / jnp.float32(total_n)
        coeff = jnp.where(
            (tp == 0) & (fn == 0),
            jnp.float32(1.0),
            jnp.where(tp == 0, jnp.float32(0.0), tp / (tp + fn)),
        )
        out_ref[0] = mse + _LAMBD * (-jnp.log(coeff + _EPS))


def kernel(outputs, labels):
    rows, cols = outputs.shape
    g = (cols + _BC - 1) // _BC
    body = functools.partial(
        _loss_kernel, total_cols=cols, total_n=outputs.size, grid_len=g)
    out = pl.pallas_call(
        body,
        grid=(g,),
        in_specs=[
            pl.BlockSpec((rows, _BC), lambda i: (0, i)),
            pl.BlockSpec((rows, _BC), lambda i: (0, i)),
        ],
        out_specs=pl.BlockSpec(memory_space=pltpu.SMEM),
        out_shape=jax.ShapeDtypeStruct((1,), jnp.float32),
        scratch_shapes=[
            pltpu.VMEM((1, _BC), jnp.float32),
            pltpu.VMEM((1, _BC), jnp.float32),
            pltpu.VMEM((1, _BC), jnp.float32),
        ],
        compiler_params=pltpu.CompilerParams(
            dimension_semantics=("arbitrary",),
            vmem_limit_bytes=56 * 1024 * 1024,
        ),
        name="detection_loss_fused",
    )(outputs, labels)
    return out[0]
